# baseline (speedup 1.0000x reference)
import functools

import jax
import jax.numpy as jnp
from jax import lax
from jax.experimental import pallas as pl
from jax.experimental.pallas import tpu as pltpu
from jax.experimental.pallas import tpu_sc as plsc

_NC = 1
_NS = 16
_NW = _NC * _NS
_L = 16


@functools.lru_cache(maxsize=None)
def _build(B, W0, W1, R0, R1):
    cols = B // _NW
    n_vec = cols // _L

    mesh = plsc.VectorSubcoreMesh(
        core_axis_name="c", subcore_axis_name="s",
        num_cores=_NC, num_subcores=_NS,
    )

    @functools.partial(
        pl.kernel,
        mesh=mesh,
        out_type=jax.ShapeDtypeStruct((19, B), jnp.float32),
        compiler_params=pltpu.CompilerParams(needs_layout_passes=False),
        scratch_types=[
            pltpu.VMEM((2, cols), jnp.float32),
            pltpu.VMEM((W0, W1), jnp.float32),
            pltpu.VMEM((W0, W1), jnp.float32),
            pltpu.VMEM((R0, R1), jnp.float32),
            pltpu.VMEM((19, cols), jnp.float32),
            pltpu.SemaphoreType.DMA,
        ],
    )
    def sc_kernel(x_hbm, wh_hbm, wv_hbm, rp_hbm, out_hbm,
                  x_v, wh_v, wv_v, rp_v, out_v, sem):
        wid = lax.axis_index("s") * _NC + lax.axis_index("c")
        base = wid * cols

        c_x = pltpu.async_copy(x_hbm.at[:, pl.ds(base, cols)], x_v, sem)
        c_wh = pltpu.async_copy(wh_hbm, wh_v, sem)
        c_wv = pltpu.async_copy(wv_hbm, wv_v, sem)
        c_rp = pltpu.async_copy(rp_hbm, rp_v, sem)
        c_x.wait()
        c_wh.wait()
        c_wv.wait()
        c_rp.wait()

        def group(i):
            sl = pl.ds(i * _L, _L)
            xx = x_v[0, sl]
            yy = x_v[1, sl]
            x_idx = (xx * 20.0).astype(jnp.int32)
            y_idx = (yy * 20.0).astype(jnp.int32)

            wh = plsc.load_gather(wh_v, [x_idx, y_idx])
            wv = plsc.load_gather(wv_v, [x_idx, y_idx])

            rx = x_idx >> 2
            ry = y_idx >> 2
            rp_c = plsc.load_gather(rp_v, [rx, ry])
            p_l = plsc.load_gather(rp_v, [jnp.maximum(rx - 1, 0), ry])
            p_r = plsc.load_gather(rp_v, [jnp.minimum(rx + 1, R0 - 1), ry])
            p_t = plsc.load_gather(rp_v, [rx, jnp.maximum(ry - 1, 0)])
            p_b = plsc.load_gather(rp_v, [rx, jnp.minimum(ry + 1, R1 - 1)])

            xm = (x_idx & 3).astype(jnp.float32)
            ym = (y_idx & 3).astype(jnp.float32)
            x_in = x_idx != (W0 - 1)
            y_in = y_idx != (W1 - 1)
            d_l = jnp.where(x_in, xm / 20.0, 0.2)
            d_r = jnp.where(x_in, (4.0 - xm) / 20.0, 0.0)
            d_t = jnp.where(y_in, (4.0 - ym) / 20.0, 0.0)
            d_b = jnp.where(y_in, ym / 20.0, 0.2)

            cols_vals = (
                xx, yy, 0.95 - xx, 0.95 - yy, wh, wv, rp_c,
                xx, 1.0 - xx, yy, 1.0 - yy,
                p_l, p_r, p_t, p_b,
                d_l, d_r, d_t, d_b,
            )
            for c, val in enumerate(cols_vals):
                out_v[c, sl] = val

        unroll = 4

        def body(it, carry):
            for u in range(unroll):
                group(it * unroll + u)
            return carry

        lax.fori_loop(0, n_vec // unroll, body, 0)

        pltpu.sync_copy(out_v, out_hbm.at[:, pl.ds(base, cols)])

    return sc_kernel


def kernel(x, wind_map_horizontal, wind_map_vertical, region_penalty_map):
    B = x.shape[0]
    W0, W1 = wind_map_horizontal.shape
    R0, R1 = region_penalty_map.shape
    fn = _build(B, W0, W1, R0, R1)
    out = fn(x.T, wind_map_horizontal, wind_map_vertical,
             region_penalty_map)
    return out.T


# speedup vs baseline: 1.0166x; 1.0166x over previous
import functools

import jax
import jax.numpy as jnp
from jax import lax
from jax.experimental import pallas as pl
from jax.experimental.pallas import tpu as pltpu
from jax.experimental.pallas import tpu_sc as plsc

_NC = 1
_NS = 16
_NW = _NC * _NS
_L = 16


@functools.lru_cache(maxsize=None)
def _build(B, W0, W1, R0, R1):
    cols = B // _NW
    n_vec = cols // _L

    mesh = plsc.VectorSubcoreMesh(
        core_axis_name="c", subcore_axis_name="s",
        num_cores=_NC, num_subcores=_NS,
    )

    @functools.partial(
        pl.kernel,
        mesh=mesh,
        out_type=jax.ShapeDtypeStruct((19, B), jnp.float32),
        compiler_params=pltpu.CompilerParams(needs_layout_passes=False),
        scratch_types=[
            pltpu.VMEM((2, cols), jnp.float32),
            pltpu.VMEM((W0, W1), jnp.float32),
            pltpu.VMEM((W0, W1), jnp.float32),
            pltpu.VMEM((R0, R1), jnp.float32),
            pltpu.VMEM((19, cols), jnp.float32),
            pltpu.SemaphoreType.DMA,
        ],
    )
    def sc_kernel(x_hbm, wh_hbm, wv_hbm, rp_hbm, out_hbm,
                  x_v, wh_v, wv_v, rp_v, out_v, sem):
        wid = lax.axis_index("s") * _NC + lax.axis_index("c")
        base = wid * cols

        c_x = pltpu.async_copy(x_hbm.at[:, pl.ds(base, cols)], x_v, sem)
        c_wh = pltpu.async_copy(wh_hbm, wh_v, sem)
        c_wv = pltpu.async_copy(wv_hbm, wv_v, sem)
        c_rp = pltpu.async_copy(rp_hbm, rp_v, sem)
        c_x.wait()
        c_wh.wait()
        c_wv.wait()
        c_rp.wait()

        def group(i):
            sl = pl.ds(i * _L, _L)
            xx = x_v[0, sl]
            yy = x_v[1, sl]
            x_idx = (xx * 20.0).astype(jnp.int32)
            y_idx = (yy * 20.0).astype(jnp.int32)

            wh = plsc.load_gather(wh_v, [x_idx, y_idx])
            wv = plsc.load_gather(wv_v, [x_idx, y_idx])

            rx = x_idx >> 2
            ry = y_idx >> 2
            rp_c = plsc.load_gather(rp_v, [rx, ry])
            p_l = plsc.load_gather(rp_v, [jnp.maximum(rx - 1, 0), ry])
            p_r = plsc.load_gather(rp_v, [jnp.minimum(rx + 1, R0 - 1), ry])
            p_t = plsc.load_gather(rp_v, [rx, jnp.maximum(ry - 1, 0)])
            p_b = plsc.load_gather(rp_v, [rx, jnp.minimum(ry + 1, R1 - 1)])

            xm = (x_idx & 3).astype(jnp.float32)
            ym = (y_idx & 3).astype(jnp.float32)
            x_in = x_idx != (W0 - 1)
            y_in = y_idx != (W1 - 1)
            d_l = jnp.where(x_in, xm / 20.0, 0.2)
            d_r = jnp.where(x_in, (4.0 - xm) / 20.0, 0.0)
            d_t = jnp.where(y_in, (4.0 - ym) / 20.0, 0.0)
            d_b = jnp.where(y_in, ym / 20.0, 0.2)

            cols_vals = (
                xx, yy, 0.95 - xx, 0.95 - yy, wh, wv, rp_c,
                xx, 1.0 - xx, yy, 1.0 - yy,
                p_l, p_r, p_t, p_b,
                d_l, d_r, d_t, d_b,
            )
            for c, val in enumerate(cols_vals):
                out_v[c, sl] = val

        unroll = 2

        def body(it, carry):
            for u in range(unroll):
                group(it * unroll + u)
            return carry

        half = cols // 2
        lax.fori_loop(0, n_vec // unroll // 2, body, 0)
        c_o1 = pltpu.async_copy(
            out_v.at[:, pl.ds(0, half)],
            out_hbm.at[:, pl.ds(base, half)], sem)
        lax.fori_loop(n_vec // unroll // 2, n_vec // unroll, body, 0)
        c_o2 = pltpu.async_copy(
            out_v.at[:, pl.ds(half, half)],
            out_hbm.at[:, pl.ds(base + half, half)], sem)
        c_o1.wait()
        c_o2.wait()

    return sc_kernel


def kernel(x, wind_map_horizontal, wind_map_vertical, region_penalty_map):
    B = x.shape[0]
    W0, W1 = wind_map_horizontal.shape
    R0, R1 = region_penalty_map.shape
    fn = _build(B, W0, W1, R0, R1)
    out = fn(x.T, wind_map_horizontal, wind_map_vertical,
             region_penalty_map)
    return out.T


# speedup vs baseline: 1.0293x; 1.0125x over previous
import functools

import jax
import jax.numpy as jnp
from jax import lax
from jax.experimental import pallas as pl
from jax.experimental.pallas import tpu as pltpu
from jax.experimental.pallas import tpu_sc as plsc

_NC = 1
_NS = 16
_NW = _NC * _NS
_L = 16


@functools.lru_cache(maxsize=None)
def _build(B, W0, W1, R0, R1):
    cols = B // _NW
    n_vec = cols // _L

    mesh = plsc.VectorSubcoreMesh(
        core_axis_name="c", subcore_axis_name="s",
        num_cores=_NC, num_subcores=_NS,
    )

    @functools.partial(
        pl.kernel,
        mesh=mesh,
        out_type=jax.ShapeDtypeStruct((19, B), jnp.float32),
        compiler_params=pltpu.CompilerParams(needs_layout_passes=False),
        scratch_types=[
            pltpu.VMEM((2, cols), jnp.float32),
            pltpu.VMEM((W0, W1), jnp.float32),
            pltpu.VMEM((W0, W1), jnp.float32),
            pltpu.VMEM((R0, R1), jnp.float32),
            pltpu.VMEM((19, cols), jnp.float32),
            pltpu.VMEM((W0 * W1 + _L,), jnp.float32),
            pltpu.VMEM((W0 * W1 + _L,), jnp.float32),
            pltpu.VMEM((R0 * R1 + _L,), jnp.float32),
            pltpu.SemaphoreType.DMA,
        ],
    )
    def sc_kernel(x_hbm, wh_hbm, wv_hbm, rp_hbm, out_hbm,
                  x_v, wh_v, wv_v, rp_v, out_v,
                  whf, wvf, rpf, sem):
        wid = lax.axis_index("s") * _NC + lax.axis_index("c")
        base = wid * cols

        c_x = pltpu.async_copy(x_hbm.at[:, pl.ds(base, cols)], x_v, sem)
        c_wh = pltpu.async_copy(wh_hbm, wh_v, sem)
        c_wv = pltpu.async_copy(wv_hbm, wv_v, sem)
        c_rp = pltpu.async_copy(rp_hbm, rp_v, sem)
        c_x.wait()
        c_wh.wait()
        c_wv.wait()
        c_rp.wait()

        lane = lax.iota(jnp.int32, _L)
        for src, dst in ((wh_v, whf), (wv_v, wvf)):
            for r in range(W0):
                dst[pl.ds(r * W1, _L)] = src[r, pl.ds(0, _L)]
                dst[pl.ds(r * W1 + (W1 - _L), _L)] = \
                    src[r, pl.ds(W1 - _L, _L)]
        for r in range(R0):
            vals = plsc.load_gather(
                rp_v, [jnp.full((_L,), r, jnp.int32),
                       jnp.minimum(lane, R1 - 1)])
            plsc.store_scatter(rpf, [r * R1 + lane], vals,
                               mask=lane < R1)

        def group(i):
            sl = pl.ds(i * _L, _L)
            xx = x_v[0, sl]
            yy = x_v[1, sl]
            x_idx = (xx * 20.0).astype(jnp.int32)
            y_idx = (yy * 20.0).astype(jnp.int32)

            wflat = x_idx * W1 + y_idx
            wh = plsc.load_gather(whf, [wflat])
            wv = plsc.load_gather(wvf, [wflat])

            rx = x_idx >> 2
            ry = y_idx >> 2
            rflat = rx * R1 + ry
            rp_c = plsc.load_gather(rpf, [rflat])
            p_l = plsc.load_gather(
                rpf, [jnp.maximum(rflat - R1, ry)])
            p_r = plsc.load_gather(
                rpf, [jnp.minimum(rflat + R1, ry + (R0 - 1) * R1)])
            p_t = plsc.load_gather(rpf, [jnp.maximum(rflat - 1, rx * R1)])
            p_b = plsc.load_gather(
                rpf, [jnp.minimum(rflat + 1, rx * R1 + (R1 - 1))])

            xm = (x_idx & 3).astype(jnp.float32)
            ym = (y_idx & 3).astype(jnp.float32)
            x_in = x_idx != (W0 - 1)
            y_in = y_idx != (W1 - 1)
            d_l = jnp.where(x_in, xm / 20.0, 0.2)
            d_r = jnp.where(x_in, (4.0 - xm) / 20.0, 0.0)
            d_t = jnp.where(y_in, (4.0 - ym) / 20.0, 0.0)
            d_b = jnp.where(y_in, ym / 20.0, 0.2)

            cols_vals = (
                xx, yy, 0.95 - xx, 0.95 - yy, wh, wv, rp_c,
                xx, 1.0 - xx, yy, 1.0 - yy,
                p_l, p_r, p_t, p_b,
                d_l, d_r, d_t, d_b,
            )
            for c, val in enumerate(cols_vals):
                out_v[c, sl] = val

        unroll = 2

        def body(it, carry):
            for u in range(unroll):
                group(it * unroll + u)
            return carry

        half = cols // 2
        lax.fori_loop(0, n_vec // unroll // 2, body, 0)
        c_o1 = pltpu.async_copy(
            out_v.at[:, pl.ds(0, half)],
            out_hbm.at[:, pl.ds(base, half)], sem)
        lax.fori_loop(n_vec // unroll // 2, n_vec // unroll, body, 0)
        c_o2 = pltpu.async_copy(
            out_v.at[:, pl.ds(half, half)],
            out_hbm.at[:, pl.ds(base + half, half)], sem)
        c_o1.wait()
        c_o2.wait()

    return sc_kernel


def kernel(x, wind_map_horizontal, wind_map_vertical, region_penalty_map):
    B = x.shape[0]
    W0, W1 = wind_map_horizontal.shape
    R0, R1 = region_penalty_map.shape
    fn = _build(B, W0, W1, R0, R1)
    out = fn(x.T, wind_map_horizontal, wind_map_vertical,
             region_penalty_map)
    return out.T
